# Initial kernel scaffold; baseline (speedup 1.0000x reference)
import jax
import jax.numpy as jnp
from jax.experimental import pallas as pl


def kernel(user_ids, pos_movie_ids, neg_movie_ids, user_emb_raw, movie_emb_raw, user_proj_w, user_proj_b, movie_proj_w, movie_proj_b, user_ws, user_bs, movie_ws, movie_bs, neighbor_table):
    raise NotImplementedError("write your pallas kernel here")


# speedup vs baseline: 3.5084x; 3.5084x over previous
import functools

import jax
import jax.numpy as jnp
from jax import lax
from jax.experimental import pallas as pl
from jax.experimental.pallas import tpu as pltpu
from jax.experimental.pallas import tpu_sc as plsc

NC, NS = 2, 16
NW = NC * NS
B = 4096
DEG = 32
D = 256
RPW = B // NW


def _proj_body(x_ref, w_ref, b_ref, o_ref):
    acc = lax.dot_general(x_ref[...], w_ref[...], (((1,), (1,)), ((), ())),
                          preferred_element_type=jnp.float32)
    o_ref[...] = jnp.maximum(acc + b_ref[...], 0.0)


def _project_table(x, w, b, bm):
    n, feat = x.shape
    return pl.pallas_call(
        _proj_body,
        grid=(n // bm,),
        in_specs=[
            pl.BlockSpec((bm, feat), lambda i: (i, 0)),
            pl.BlockSpec(w.shape, lambda i: (0, 0)),
            pl.BlockSpec((1, D), lambda i: (0, 0)),
        ],
        out_specs=pl.BlockSpec((bm, D), lambda i: (i, 0)),
        out_shape=jax.ShapeDtypeStruct((n, D), jnp.float32),
    )(x, w, b.reshape(1, D))


def _sc_body(movie_emb, uraw_tab, ntabq, uids, pids, nids, m2,
             agg_o, pos_o, neg_o, uraw_o,
             uid_v, pid_v, nid_v, uidq_v, m2_v, gath_v, rows0, rows1, accb,
             sem0, sem1, semg):
    wid = lax.axis_index("s") * NC + lax.axis_index("c")
    base = wid * RPW

    pltpu.sync_copy(uids.at[pl.ds(base, RPW)], uid_v)
    pltpu.sync_copy(pids.at[pl.ds(base, RPW)], pid_v)
    pltpu.sync_copy(nids.at[pl.ds(base, RPW)], nid_v)
    pltpu.sync_copy(m2.at[pl.ds(base, RPW)], m2_v)

    for c in range(RPW // 16):
        v = uid_v[pl.ds(c * 16, 16)]
        uidq_v[pl.ds(c * 16, 16)] = lax.shift_right_logical(v, 2)
    pltpu.async_copy(ntabq.at[uidq_v], gath_v, semg).wait()

    def extract(p, carry):
        for sub in range(2):
            i = 2 * p + sub
            ms = m2_v[i, pl.ds(0, 16)]
            for c in range(DEG // 16):
                cands = [gath_v[i, pl.ds(k * DEG + c * 16, 16)]
                         for k in range(4)]
                r = jnp.where(ms == 0, cands[0],
                    jnp.where(ms == 1, cands[1],
                    jnp.where(ms == 2, cands[2], cands[3])))
                gath_v[p, pl.ds(sub * DEG + c * 16, 16)] = r
        return carry

    lax.fori_loop(0, RPW // 2, extract, 0)
    pltpu.async_copy(uraw_tab.at[uid_v], accb, semg).wait()
    pltpu.sync_copy(accb, uraw_o.at[pl.ds(base, RPW)])
    pltpu.async_copy(movie_emb.at[pid_v], accb, semg).wait()
    pltpu.sync_copy(accb, pos_o.at[pl.ds(base, RPW)])
    pltpu.async_copy(movie_emb.at[nid_v], accb, semg).wait()
    pltpu.sync_copy(accb, neg_o.at[pl.ds(base, RPW)])

    NP = RPW // 2

    def nbr_idx(p):
        return gath_v.at[p, pl.ds(0, 2 * DEG)]

    pltpu.async_copy(movie_emb.at[nbr_idx(0)], rows0, sem0)
    pltpu.async_copy(movie_emb.at[nbr_idx(1)], rows1, sem1)

    def acc_pair(rv, p):
        for sub in range(2):
            for c in range(D // 16):
                sl = pl.ds(c * 16, 16)
                a = [rv[sub * DEG + l, sl] for l in range(8)]
                for j in range(8, DEG, 8):
                    for l in range(8):
                        a[l] = a[l] + rv[sub * DEG + j + l, sl]
                s0 = (a[0] + a[1]) + (a[2] + a[3])
                s1 = (a[4] + a[5]) + (a[6] + a[7])
                accb[2 * p + sub, sl] = (s0 + s1) * (1.0 / DEG)

    def step(k, carry):
        p0 = 2 * k
        pltpu.make_async_copy(movie_emb.at[nbr_idx(p0)], rows0, sem0).wait()
        acc_pair(rows0, p0)

        @pl.when(p0 + 2 < NP)
        def _():
            pltpu.async_copy(movie_emb.at[nbr_idx(p0 + 2)], rows0, sem0)

        pltpu.make_async_copy(movie_emb.at[nbr_idx(p0 + 1)], rows1, sem1).wait()
        acc_pair(rows1, p0 + 1)

        @pl.when(p0 + 3 < NP)
        def _():
            pltpu.async_copy(movie_emb.at[nbr_idx(p0 + 3)], rows1, sem1)

        return carry

    lax.fori_loop(0, NP // 2, step, 0)
    pltpu.sync_copy(accb, agg_o.at[pl.ds(base, RPW)])


def _sc_gather(movie_emb, uraw_tab, ntabq, uids, pids, nids, m2):
    out_t = jax.ShapeDtypeStruct((B, D), jnp.float32)
    fn = pl.kernel(
        _sc_body,
        out_type=[out_t, out_t, out_t, out_t],
        mesh=plsc.VectorSubcoreMesh(core_axis_name="c", subcore_axis_name="s"),
        scratch_types=[
            pltpu.VMEM((RPW,), jnp.int32),
            pltpu.VMEM((RPW,), jnp.int32),
            pltpu.VMEM((RPW,), jnp.int32),
            pltpu.VMEM((RPW,), jnp.int32),
            pltpu.VMEM((RPW, 16), jnp.int32),
            pltpu.VMEM((RPW, 4 * DEG), jnp.int32),
            pltpu.VMEM((2 * DEG, D), jnp.float32),
            pltpu.VMEM((2 * DEG, D), jnp.float32),
            pltpu.VMEM((RPW, D), jnp.float32),
            pltpu.SemaphoreType.DMA,
            pltpu.SemaphoreType.DMA,
            pltpu.SemaphoreType.DMA,
        ],
    )
    return fn(movie_emb, uraw_tab, ntabq, uids, pids, nids, m2)


def _mlp_body(uraw_ref, agg_ref, pos_ref, neg_ref,
              upw_ref, upb_ref, uws_ref, ubs_ref, mws_ref, mbs_ref,
              users_ref, pos_o_ref, neg_o_ref):
    nl = uws_ref.shape[0]

    def mm(x, w):
        return lax.dot_general(x, w, (((1,), (1,)), ((), ())),
                               preferred_element_type=jnp.float32)

    u0 = jnp.maximum(mm(uraw_ref[...], upw_ref[...]) + upb_ref[...], 0.0)
    bu = agg_ref[...]
    acc_u = u0
    p = pos_ref[...]
    ng = neg_ref[...]
    acc_p = p
    acc_n = ng
    for i in range(nl):
        bu = jnp.maximum(mm(bu, uws_ref[i]) + ubs_ref[i], 0.0)
        acc_u = acc_u + bu
        p = jnp.maximum(mm(p, mws_ref[i]) + mbs_ref[i], 0.0)
        acc_p = acc_p + p
        ng = jnp.maximum(mm(ng, mws_ref[i]) + mbs_ref[i], 0.0)
        acc_n = acc_n + ng
    scale = 1.0 / (nl + 1)
    users_ref[...] = acc_u * scale
    pos_o_ref[...] = acc_p * scale
    neg_o_ref[...] = acc_n * scale


def _mlp(uraw, agg, pos0, neg0, upw, upb, uws, ubs, mws, mbs, bm):
    nl = uws.shape[0]
    full = lambda s: pl.BlockSpec(s, lambda i: tuple(0 for _ in s))
    out_t = jax.ShapeDtypeStruct((B, D), jnp.float32)
    row_spec = pl.BlockSpec((bm, D), lambda i: (i, 0))
    return pl.pallas_call(
        _mlp_body,
        grid=(B // bm,),
        in_specs=[
            row_spec, row_spec, row_spec, row_spec,
            full((D, D)), full((1, D)),
            full((nl, D, D)), full((nl, 1, D)),
            full((nl, D, D)), full((nl, 1, D)),
        ],
        out_specs=[row_spec, row_spec, row_spec],
        out_shape=[out_t, out_t, out_t],
    )(uraw, agg, pos0, neg0, upw, upb.reshape(1, D),
      uws, ubs.reshape(nl, 1, D), mws, mbs.reshape(nl, 1, D))


def kernel(user_ids, pos_movie_ids, neg_movie_ids, user_emb_raw, movie_emb_raw,
           user_proj_w, user_proj_b, movie_proj_w, movie_proj_b,
           user_ws, user_bs, movie_ws, movie_bs, neighbor_table):
    uids = user_ids.astype(jnp.int32)
    pids = pos_movie_ids.astype(jnp.int32)
    nids = neg_movie_ids.astype(jnp.int32)
    ntabq = neighbor_table.astype(jnp.int32).reshape(-1, 4 * DEG)

    m2 = jnp.broadcast_to((uids & 3)[:, None], (B, 16)).astype(jnp.int32)

    movie_emb = _project_table(movie_emb_raw, movie_proj_w, movie_proj_b, 1000)
    agg, pos0, neg0, uraw = _sc_gather(movie_emb, user_emb_raw, ntabq,
                                       uids, pids, nids, m2)
    users, pos_out, neg_out = _mlp(uraw, agg, pos0, neg0,
                                   user_proj_w, user_proj_b,
                                   user_ws, user_bs, movie_ws, movie_bs, 512)
    return users, pos_out, neg_out


# speedup vs baseline: 4.4046x; 1.2554x over previous
import functools

import jax
import jax.numpy as jnp
from jax import lax
from jax.experimental import pallas as pl
from jax.experimental.pallas import tpu as pltpu
from jax.experimental.pallas import tpu_sc as plsc

NC, NS = 2, 16
NW = NC * NS
B = 4096
DEG = 32
D = 256
RPW = B // NW


def _proj_body(x_ref, w_ref, b_ref, o_ref):
    xb = x_ref[...].astype(jnp.bfloat16)
    acc = lax.dot_general(xb, w_ref[...], (((1,), (1,)), ((), ())),
                          preferred_element_type=jnp.float32)
    o_ref[...] = jnp.maximum(acc + b_ref[...], 0.0)


def _project_table(x, w, b, bm):
    n, feat = x.shape
    return pl.pallas_call(
        _proj_body,
        grid=(n // bm,),
        in_specs=[
            pl.BlockSpec((bm, feat), lambda i: (i, 0)),
            pl.BlockSpec(w.shape, lambda i: (0, 0)),
            pl.BlockSpec((1, D), lambda i: (0, 0)),
        ],
        out_specs=pl.BlockSpec((bm, D), lambda i: (i, 0)),
        out_shape=jax.ShapeDtypeStruct((n, D), jnp.float32),
    )(x, w, b.reshape(1, D))


def _sc_body(movie_emb, uraw_tab, ntabq, uids, pids, nids, m2,
             agg_o, pos_o, neg_o, uraw_o,
             uid_v, pid_v, nid_v, uidq_v, m2_v, gath_v, rows0, rows1, accb,
             sem0, sem1, semg):
    wid = lax.axis_index("s") * NC + lax.axis_index("c")
    base = wid * RPW

    pltpu.sync_copy(uids.at[pl.ds(base, RPW)], uid_v)
    pltpu.sync_copy(pids.at[pl.ds(base, RPW)], pid_v)
    pltpu.sync_copy(nids.at[pl.ds(base, RPW)], nid_v)
    pltpu.sync_copy(m2.at[pl.ds(base, RPW)], m2_v)

    for c in range(RPW // 16):
        v = uid_v[pl.ds(c * 16, 16)]
        uidq_v[pl.ds(c * 16, 16)] = lax.shift_right_logical(v, 2)
    pltpu.async_copy(ntabq.at[uidq_v], gath_v, semg).wait()

    def extract(i, carry):
        ms = m2_v[i, pl.ds(0, 16)]
        for c in range(DEG // 16):
            cands = [gath_v[i, pl.ds(k * DEG + c * 16, 16)]
                     for k in range(4)]
            r = jnp.where(ms == 0, cands[0],
                jnp.where(ms == 1, cands[1],
                jnp.where(ms == 2, cands[2], cands[3])))
            gath_v[i, pl.ds(c * 16, 16)] = r
        return carry

    lax.fori_loop(0, RPW, extract, 0)
    pltpu.async_copy(uraw_tab.at[uid_v], accb, semg).wait()
    pltpu.sync_copy(accb, uraw_o.at[pl.ds(base, RPW)])
    pltpu.async_copy(movie_emb.at[pid_v], accb, semg).wait()
    pltpu.sync_copy(accb, pos_o.at[pl.ds(base, RPW)])
    pltpu.async_copy(movie_emb.at[nid_v], accb, semg).wait()
    pltpu.sync_copy(accb, neg_o.at[pl.ds(base, RPW)])

    NP = RPW

    def nbr_idx(p):
        return gath_v.at[p, pl.ds(0, DEG)]

    pltpu.async_copy(movie_emb.at[nbr_idx(0)], rows0, sem0)
    pltpu.async_copy(movie_emb.at[nbr_idx(1)], rows1, sem1)

    def acc_pair(rv, p):
        for c in range(D // 16):
            sl = pl.ds(c * 16, 16)
            a = [rv[l, sl] for l in range(4)]
            for j in range(4, DEG, 4):
                for l in range(4):
                    a[l] = a[l] + rv[j + l, sl]
            accb[p, sl] = ((a[0] + a[1]) + (a[2] + a[3])) * (1.0 / DEG)

    def step(k, carry):
        p0 = 2 * k
        pltpu.make_async_copy(movie_emb.at[nbr_idx(p0)], rows0, sem0).wait()
        acc_pair(rows0, p0)

        @pl.when(p0 + 2 < NP)
        def _():
            pltpu.async_copy(movie_emb.at[nbr_idx(p0 + 2)], rows0, sem0)

        pltpu.make_async_copy(movie_emb.at[nbr_idx(p0 + 1)], rows1, sem1).wait()
        acc_pair(rows1, p0 + 1)

        @pl.when(p0 + 3 < NP)
        def _():
            pltpu.async_copy(movie_emb.at[nbr_idx(p0 + 3)], rows1, sem1)

        return carry

    lax.fori_loop(0, NP // 2, step, 0)
    pltpu.sync_copy(accb, agg_o.at[pl.ds(base, RPW)])


def _sc_gather(movie_emb, uraw_tab, ntabq, uids, pids, nids, m2):
    out_t = jax.ShapeDtypeStruct((B, D), jnp.float32)
    fn = pl.kernel(
        _sc_body,
        out_type=[out_t, out_t, out_t, out_t],
        mesh=plsc.VectorSubcoreMesh(core_axis_name="c", subcore_axis_name="s"),
        scratch_types=[
            pltpu.VMEM((RPW,), jnp.int32),
            pltpu.VMEM((RPW,), jnp.int32),
            pltpu.VMEM((RPW,), jnp.int32),
            pltpu.VMEM((RPW,), jnp.int32),
            pltpu.VMEM((RPW, 16), jnp.int32),
            pltpu.VMEM((RPW, 4 * DEG), jnp.int32),
            pltpu.VMEM((DEG, D), jnp.float32),
            pltpu.VMEM((DEG, D), jnp.float32),
            pltpu.VMEM((RPW, D), jnp.float32),
            pltpu.SemaphoreType.DMA,
            pltpu.SemaphoreType.DMA,
            pltpu.SemaphoreType.DMA,
        ],
    )
    return fn(movie_emb, uraw_tab, ntabq, uids, pids, nids, m2)


def _mlp_body(uraw_ref, agg_ref, pos_ref, neg_ref,
              upw_ref, upb_ref, uws_ref, ubs_ref, mws_ref, mbs_ref,
              users_ref, pos_o_ref, neg_o_ref):
    nl = uws_ref.shape[0]

    def mm(x, w):
        return lax.dot_general(x.astype(jnp.bfloat16), w,
                               (((1,), (1,)), ((), ())),
                               preferred_element_type=jnp.float32)

    u0 = jnp.maximum(mm(uraw_ref[...], upw_ref[...]) + upb_ref[...], 0.0)
    bu = agg_ref[...]
    acc_u = u0
    p = pos_ref[...]
    ng = neg_ref[...]
    acc_p = p
    acc_n = ng
    for i in range(nl):
        bu = jnp.maximum(mm(bu, uws_ref[i]) + ubs_ref[i], 0.0)
        acc_u = acc_u + bu
        p = jnp.maximum(mm(p, mws_ref[i]) + mbs_ref[i], 0.0)
        acc_p = acc_p + p
        ng = jnp.maximum(mm(ng, mws_ref[i]) + mbs_ref[i], 0.0)
        acc_n = acc_n + ng
    scale = 1.0 / (nl + 1)
    users_ref[...] = acc_u * scale
    pos_o_ref[...] = acc_p * scale
    neg_o_ref[...] = acc_n * scale


def _mlp(uraw, agg, pos0, neg0, upw, upb, uws, ubs, mws, mbs, bm):
    nl = uws.shape[0]
    full = lambda s: pl.BlockSpec(s, lambda i: tuple(0 for _ in s))
    out_t = jax.ShapeDtypeStruct((B, D), jnp.float32)
    row_spec = pl.BlockSpec((bm, D), lambda i: (i, 0))
    return pl.pallas_call(
        _mlp_body,
        grid=(B // bm,),
        in_specs=[
            row_spec, row_spec, row_spec, row_spec,
            full((D, D)), full((1, D)),
            full((nl, D, D)), full((nl, 1, D)),
            full((nl, D, D)), full((nl, 1, D)),
        ],
        out_specs=[row_spec, row_spec, row_spec],
        out_shape=[out_t, out_t, out_t],
    )(uraw, agg, pos0, neg0, upw, upb.reshape(1, D),
      uws, ubs.reshape(nl, 1, D), mws, mbs.reshape(nl, 1, D))


def kernel(user_ids, pos_movie_ids, neg_movie_ids, user_emb_raw, movie_emb_raw,
           user_proj_w, user_proj_b, movie_proj_w, movie_proj_b,
           user_ws, user_bs, movie_ws, movie_bs, neighbor_table):
    uids = user_ids.astype(jnp.int32)
    pids = pos_movie_ids.astype(jnp.int32)
    nids = neg_movie_ids.astype(jnp.int32)
    ntabq = neighbor_table.astype(jnp.int32).reshape(-1, 4 * DEG)

    m2 = jnp.broadcast_to((uids & 3)[:, None], (B, 16)).astype(jnp.int32)

    bf16 = jnp.bfloat16
    movie_emb = _project_table(movie_emb_raw, movie_proj_w.astype(bf16),
                               movie_proj_b, 1000)
    agg, pos0, neg0, uraw = _sc_gather(movie_emb, user_emb_raw, ntabq,
                                       uids, pids, nids, m2)
    users, pos_out, neg_out = _mlp(uraw, agg, pos0, neg0,
                                   user_proj_w.astype(bf16), user_proj_b,
                                   user_ws.astype(bf16), user_bs,
                                   movie_ws.astype(bf16), movie_bs, 512)
    return users, pos_out, neg_out


# speedup vs baseline: 4.4356x; 1.0070x over previous
import functools

import jax
import jax.numpy as jnp
from jax import lax
from jax.experimental import pallas as pl
from jax.experimental.pallas import tpu as pltpu
from jax.experimental.pallas import tpu_sc as plsc

NC, NS = 2, 16
NW = NC * NS
B = 4096
DEG = 32
D = 256
RPW = B // NW


def _proj_body(x_ref, w_ref, b_ref, o_ref):
    acc = lax.dot_general(x_ref[...], w_ref[...], (((1,), (1,)), ((), ())),
                          preferred_element_type=jnp.float32)
    o_ref[...] = jnp.maximum(acc + b_ref[...], 0.0)


def _project_table(x, w, b, bm):
    n, feat = x.shape
    return pl.pallas_call(
        _proj_body,
        grid=(n // bm,),
        in_specs=[
            pl.BlockSpec((bm, feat), lambda i: (i, 0)),
            pl.BlockSpec(w.shape, lambda i: (0, 0)),
            pl.BlockSpec((1, D), lambda i: (0, 0)),
        ],
        out_specs=pl.BlockSpec((bm, D), lambda i: (i, 0)),
        out_shape=jax.ShapeDtypeStruct((n, D), jnp.float32),
    )(x, w, b.reshape(1, D))


def _sc_body(movie_emb, uraw_tab, ntabq, uids, pids, nids, m2,
             agg_o, pos_o, neg_o, uraw_o,
             uid_v, pid_v, nid_v, uidq_v, m2_v, gath_v, rows0, rows1, accb,
             sem0, sem1, semg):
    wid = lax.axis_index("s") * NC + lax.axis_index("c")
    base = wid * RPW

    pltpu.sync_copy(uids.at[pl.ds(base, RPW)], uid_v)
    pltpu.sync_copy(pids.at[pl.ds(base, RPW)], pid_v)
    pltpu.sync_copy(nids.at[pl.ds(base, RPW)], nid_v)
    pltpu.sync_copy(m2.at[pl.ds(base, RPW)], m2_v)

    for c in range(RPW // 16):
        v = uid_v[pl.ds(c * 16, 16)]
        uidq_v[pl.ds(c * 16, 16)] = lax.shift_right_logical(v, 2)
    pltpu.async_copy(ntabq.at[uidq_v], gath_v, semg).wait()

    def extract(i, carry):
        ms = m2_v[i, pl.ds(0, 16)]
        for c in range(DEG // 16):
            cands = [gath_v[i, pl.ds(k * DEG + c * 16, 16)]
                     for k in range(4)]
            r = jnp.where(ms == 0, cands[0],
                jnp.where(ms == 1, cands[1],
                jnp.where(ms == 2, cands[2], cands[3])))
            gath_v[i, pl.ds(c * 16, 16)] = r
        return carry

    lax.fori_loop(0, RPW, extract, 0)
    pltpu.async_copy(uraw_tab.at[uid_v], accb, semg).wait()
    pltpu.sync_copy(accb, uraw_o.at[pl.ds(base, RPW)])
    pltpu.async_copy(movie_emb.at[pid_v], accb, semg).wait()
    pltpu.sync_copy(accb, pos_o.at[pl.ds(base, RPW)])
    pltpu.async_copy(movie_emb.at[nid_v], accb, semg).wait()
    pltpu.sync_copy(accb, neg_o.at[pl.ds(base, RPW)])

    NP = RPW

    def nbr_idx(p):
        return gath_v.at[p, pl.ds(0, DEG)]

    pltpu.async_copy(movie_emb.at[nbr_idx(0)], rows0, sem0)
    pltpu.async_copy(movie_emb.at[nbr_idx(1)], rows1, sem1)

    def acc_row(rv, p):
        for c in range(D // 16):
            sl = pl.ds(c * 16, 16)
            a = [rv[l, sl] for l in range(4)]
            for j in range(4, DEG, 4):
                for l in range(4):
                    a[l] = a[l] + rv[j + l, sl]
            accb[p, sl] = ((a[0] + a[1]) + (a[2] + a[3])) * (1.0 / DEG)

    def step(k, carry):
        p0 = 2 * k
        pltpu.make_async_copy(movie_emb.at[nbr_idx(p0)], rows0, sem0).wait()
        acc_row(rows0, p0)

        @pl.when(p0 + 2 < NP)
        def _():
            pltpu.async_copy(movie_emb.at[nbr_idx(p0 + 2)], rows0, sem0)

        pltpu.make_async_copy(movie_emb.at[nbr_idx(p0 + 1)], rows1, sem1).wait()
        acc_row(rows1, p0 + 1)

        @pl.when(p0 + 3 < NP)
        def _():
            pltpu.async_copy(movie_emb.at[nbr_idx(p0 + 3)], rows1, sem1)

        return carry

    lax.fori_loop(0, NP // 2, step, 0)
    pltpu.sync_copy(accb, agg_o.at[pl.ds(base, RPW)])


def _sc_gather(movie_emb, uraw_tab, ntabq, uids, pids, nids, m2):
    out_t = jax.ShapeDtypeStruct((B, D), jnp.float32)
    fn = pl.kernel(
        _sc_body,
        out_type=[out_t, out_t, out_t, out_t],
        mesh=plsc.VectorSubcoreMesh(core_axis_name="c", subcore_axis_name="s"),
        scratch_types=[
            pltpu.VMEM((RPW,), jnp.int32),
            pltpu.VMEM((RPW,), jnp.int32),
            pltpu.VMEM((RPW,), jnp.int32),
            pltpu.VMEM((RPW,), jnp.int32),
            pltpu.VMEM((RPW, 16), jnp.int32),
            pltpu.VMEM((RPW, 4 * DEG), jnp.int32),
            pltpu.VMEM((DEG, D), jnp.float32),
            pltpu.VMEM((DEG, D), jnp.float32),
            pltpu.VMEM((RPW, D), jnp.float32),
            pltpu.SemaphoreType.DMA,
            pltpu.SemaphoreType.DMA,
            pltpu.SemaphoreType.DMA,
        ],
    )
    return fn(movie_emb, uraw_tab, ntabq, uids, pids, nids, m2)


def _mlp_body(uraw_ref, agg_ref, pos_ref, neg_ref,
              upw_ref, upb_ref, uws_ref, ubs_ref, mws_ref, mbs_ref,
              users_ref, pos_o_ref, neg_o_ref):
    nl = uws_ref.shape[0]

    def mm(x, w):
        return lax.dot_general(x, w, (((1,), (1,)), ((), ())),
                               preferred_element_type=jnp.float32)

    u0 = jnp.maximum(mm(uraw_ref[...], upw_ref[...]) + upb_ref[...], 0.0)
    bu = agg_ref[...]
    acc_u = u0
    p = pos_ref[...]
    ng = neg_ref[...]
    acc_p = p
    acc_n = ng
    for i in range(nl):
        bu = jnp.maximum(mm(bu, uws_ref[i]) + ubs_ref[i], 0.0)
        acc_u = acc_u + bu
        p = jnp.maximum(mm(p, mws_ref[i]) + mbs_ref[i], 0.0)
        acc_p = acc_p + p
        ng = jnp.maximum(mm(ng, mws_ref[i]) + mbs_ref[i], 0.0)
        acc_n = acc_n + ng
    scale = 1.0 / (nl + 1)
    users_ref[...] = acc_u * scale
    pos_o_ref[...] = acc_p * scale
    neg_o_ref[...] = acc_n * scale


def _mlp(uraw, agg, pos0, neg0, upw, upb, uws, ubs, mws, mbs, bm):
    nl = uws.shape[0]
    full = lambda s: pl.BlockSpec(s, lambda i: tuple(0 for _ in s))
    out_t = jax.ShapeDtypeStruct((B, D), jnp.float32)
    row_spec = pl.BlockSpec((bm, D), lambda i: (i, 0))
    return pl.pallas_call(
        _mlp_body,
        grid=(B // bm,),
        in_specs=[
            row_spec, row_spec, row_spec, row_spec,
            full((D, D)), full((1, D)),
            full((nl, D, D)), full((nl, 1, D)),
            full((nl, D, D)), full((nl, 1, D)),
        ],
        out_specs=[row_spec, row_spec, row_spec],
        out_shape=[out_t, out_t, out_t],
    )(uraw, agg, pos0, neg0, upw, upb.reshape(1, D),
      uws, ubs.reshape(nl, 1, D), mws, mbs.reshape(nl, 1, D))


def kernel(user_ids, pos_movie_ids, neg_movie_ids, user_emb_raw, movie_emb_raw,
           user_proj_w, user_proj_b, movie_proj_w, movie_proj_b,
           user_ws, user_bs, movie_ws, movie_bs, neighbor_table):
    uids = user_ids.astype(jnp.int32)
    pids = pos_movie_ids.astype(jnp.int32)
    nids = neg_movie_ids.astype(jnp.int32)
    ntabq = neighbor_table.astype(jnp.int32).reshape(-1, 4 * DEG)

    m2 = jnp.broadcast_to((uids & 3)[:, None], (B, 16)).astype(jnp.int32)

    movie_emb = _project_table(movie_emb_raw, movie_proj_w, movie_proj_b, 1000)
    agg, pos0, neg0, uraw = _sc_gather(movie_emb, user_emb_raw, ntabq,
                                       uids, pids, nids, m2)
    users, pos_out, neg_out = _mlp(uraw, agg, pos0, neg0,
                                   user_proj_w, user_proj_b,
                                   user_ws, user_bs, movie_ws, movie_bs, 512)
    return users, pos_out, neg_out
